# baseline (speedup 1.0000x reference)
import functools

import jax
import jax.numpy as jnp
from jax import lax
from jax.experimental import pallas as pl
from jax.experimental.pallas import tpu as pltpu
from jax.experimental.pallas import tpu_sc as plsc

N = 10000
E = 320000
C = 128

NC = 2
NS = 16
NW = NC * NS
EPW = E // NW
BLK = 80
NB = EPW // BLK
NPAD = 10112
RPT = NPAD // NS
ZR = 8
CNT_PAD = 10240
CSTRIPE = CNT_PAD // NS


def _seg_mean_sc_body(with_counts, feat, ei, out, cnt_out, idx_v,
                      rows_v, zrow_v, ones_v, zc_v, acc_sh, cnt_sh,
                      sem_i, sem_g, sem_s, sem_c):
    cid = lax.axis_index("c")
    sid = lax.axis_index("s")
    wid = sid * NC + cid

    def zero_zrow(k, _):
        i = k // 8
        j = k - i * 8
        zrow_v[i, pl.ds(j * 16, 16)] = jnp.zeros((16,), jnp.float32)
        return 0
    lax.fori_loop(0, ZR * 8, zero_zrow, 0)

    def zero_stripe(k, _):
        pltpu.sync_copy(zrow_v, acc_sh.at[pl.ds(sid * RPT + k * ZR, ZR), :])
        return 0
    lax.fori_loop(0, RPT // ZR, zero_stripe, 0)
    if with_counts:
        def set_ones(j, _):
            ones_v[pl.ds(j * 16, 16)] = jnp.ones((16,), jnp.float32)
            return 0
        lax.fori_loop(0, BLK // 16, set_ones, 0)

        def zero_zc(j, _):
            zc_v[pl.ds(j * 16, 16)] = jnp.zeros((16,), jnp.float32)
            return 0
        lax.fori_loop(0, CSTRIPE // 16, zero_zc, 0)
        pltpu.sync_copy(zc_v, cnt_sh.at[pl.ds(sid * CSTRIPE, CSTRIPE)])
    plsc.subcore_barrier()

    def start_idx(j, s):
        pltpu.async_copy(ei.at[wid, j], idx_v.at[s], sem_i.at[s])

    def wait_idx(j, s):
        pltpu.make_async_copy(ei.at[wid, j], idx_v.at[s],
                              sem_i.at[s]).wait()

    def start_gather(b, s):
        pltpu.async_copy(feat.at[idx_v.at[s, 0]], rows_v.at[b], sem_g.at[b])

    def wait_gather(b, s):
        pltpu.make_async_copy(feat.at[idx_v.at[s, 0]], rows_v.at[b],
                              sem_g.at[b]).wait()

    def start_scatter(b, s):
        pltpu.async_copy(rows_v.at[b], acc_sh.at[idx_v.at[s, 1]],
                         sem_s.at[b], add=True)
        if with_counts:
            pltpu.async_copy(ones_v, cnt_sh.at[idx_v.at[s, 1]],
                             sem_c.at[b], add=True)

    def wait_scatter(b, s):
        pltpu.make_async_copy(rows_v.at[b], acc_sh.at[idx_v.at[s, 1]],
                              sem_s.at[b]).wait()
        if with_counts:
            pltpu.make_async_copy(ones_v, cnt_sh.at[idx_v.at[s, 1]],
                                  sem_c.at[b]).wait()

    for t in range(8):
        start_idx(t, t)
    for t in range(3):
        wait_idx(t, t)
        start_gather(t, t)
    wait_gather(0, 0)
    start_scatter(0, 0)
    wait_idx(3, 3)
    start_gather(3, 3)
    wait_gather(1, 1)
    start_scatter(1, 1)

    def edge_oct(k, _):
        for u in range(8):
            j = 8 * k + 2 + u
            s = (2 + u) % 8
            b = (2 + u) % 4
            wait_scatter((b - 2) % 4, (s - 2) % 8)
            pltpu.async_copy(ei.at[wid, jnp.minimum(j + 6, NB - 1)],
                             idx_v.at[(s + 6) % 8], sem_i.at[(s + 6) % 8])
            wait_idx(j + 2, (s + 2) % 8)
            start_gather((b + 2) % 4, (s + 2) % 8)
            wait_gather(b, s)
            start_scatter(b, s)
        return 0
    assert (NB - 5) % 8 == 0
    lax.fori_loop(0, (NB - 5) // 8, edge_oct, 0)
    wait_scatter(0, 0)
    wait_idx(NB - 1, 4)
    start_gather(0, 4)
    wait_gather(2, 2)
    start_scatter(2, 2)
    wait_scatter(1, 1)
    wait_gather(3, 3)
    start_scatter(3, 3)
    wait_scatter(2, 2)
    wait_gather(0, 4)
    start_scatter(0, 4)
    wait_scatter(3, 3)
    wait_scatter(0, 4)
    wait_idx(NB - 1, 5)
    wait_idx(NB - 1, 6)
    wait_idx(NB - 1, 7)
    plsc.subcore_barrier()

    pltpu.sync_copy(acc_sh.at[pl.ds(sid * RPT, RPT), :],
                    out.at[cid, pl.ds(sid * RPT, RPT), :])
    if with_counts:
        pltpu.sync_copy(cnt_sh.at[pl.ds(sid * CSTRIPE, CSTRIPE)],
                        cnt_out.at[cid, pl.ds(sid * CSTRIPE, CSTRIPE)])


@functools.lru_cache(maxsize=None)
def _make_seg_sum_sc(with_counts):
    mesh = plsc.VectorSubcoreMesh(core_axis_name="c", subcore_axis_name="s",
                                  num_cores=NC, num_subcores=NS)
    out_type = [jax.ShapeDtypeStruct((NC, NPAD, C), jnp.float32)]
    if with_counts:
        out_type.append(jax.ShapeDtypeStruct((NC, CNT_PAD), jnp.float32))

    def body(*refs):
        if with_counts:
            feat, ei, out, cnt_out, idx_v, rows_v, zrow_v, ones_v, \
                zc_v, acc_sh, cnt_sh, sem_i, sem_g, sem_s, sem_c = refs
        else:
            feat, ei, out, idx_v, rows_v, zrow_v, acc_sh, \
                sem_i, sem_g, sem_s = refs
            cnt_out = ones_v = zc_v = cnt_sh = sem_c = None
        _seg_mean_sc_body(with_counts, feat, ei, out, cnt_out, idx_v,
                          rows_v, zrow_v, ones_v, zc_v, acc_sh, cnt_sh,
                          sem_i, sem_g, sem_s, sem_c)

    scratch = [
        pltpu.VMEM((8, 2, BLK), jnp.int32),
        pltpu.VMEM((4, BLK, C), jnp.float32),
        pltpu.VMEM((ZR, C), jnp.float32),
    ]
    if with_counts:
        scratch.append(pltpu.VMEM((BLK,), jnp.float32))
        scratch.append(pltpu.VMEM((CSTRIPE,), jnp.float32))
    scratch.append(pltpu.VMEM_SHARED((NPAD, C), jnp.float32))
    if with_counts:
        scratch.append(pltpu.VMEM_SHARED((CNT_PAD,), jnp.float32))
    scratch.append(pltpu.SemaphoreType.DMA((8,)))
    scratch.append(pltpu.SemaphoreType.DMA((4,)))
    scratch.append(pltpu.SemaphoreType.DMA((4,)))
    if with_counts:
        scratch.append(pltpu.SemaphoreType.DMA((4,)))

    return pl.kernel(body, out_type=tuple(out_type), mesh=mesh,
                     scratch_types=tuple(scratch),
                     name=f"seg_sum_sc{'_cnt' if with_counts else ''}")


RB = 1000
NRB = N // RB


def _combine_matmul_body(p_ref, c_ref, w_ref, b_ref, o_ref):
    s = p_ref[0] + p_ref[1]
    c = c_ref[0] + c_ref[1]
    agg = s / jnp.maximum(c, 1.0)
    o_ref[...] = jnp.dot(agg, w_ref[...],
                         preferred_element_type=jnp.float32) + b_ref[...]


def _combine_matmul(p, cnt, w, b):
    return pl.pallas_call(
        _combine_matmul_body,
        grid=(NRB,),
        in_specs=[
            pl.BlockSpec((NC, RB, C), lambda i: (0, i, 0)),
            pl.BlockSpec((NC, RB, 1), lambda i: (0, i, 0)),
            pl.BlockSpec((C, C), lambda i: (0, 0)),
            pl.BlockSpec((1, C), lambda i: (0, 0)),
        ],
        out_specs=pl.BlockSpec((RB, C), lambda i: (i, 0)),
        out_shape=jax.ShapeDtypeStruct((N, C), jnp.float32),
    )(p, cnt, w, b)


def _stats_body(p_ref, c_ref, w_ref, b_ref, mu_ref, var_ref, acc):
    i = pl.program_id(0)

    @pl.when(i == 0)
    def _():
        acc[...] = jnp.zeros_like(acc)

    s = p_ref[0] + p_ref[1]
    c = c_ref[0] + c_ref[1]
    agg = s / jnp.maximum(c, 1.0)
    h = jnp.dot(agg, w_ref[...], preferred_element_type=jnp.float32) \
        + b_ref[...]
    acc[0:1, :] += jnp.sum(h, axis=0, keepdims=True)
    acc[1:2, :] += jnp.sum(h * h, axis=0, keepdims=True)

    @pl.when(i == NRB - 1)
    def _():
        mu = acc[0:1, :] * (1.0 / N)
        mu_ref[...] = mu
        var_ref[...] = acc[1:2, :] * (1.0 / N) - mu * mu


def _h2_stats(p, cnt, w, b):
    return pl.pallas_call(
        _stats_body,
        grid=(NRB,),
        in_specs=[
            pl.BlockSpec((NC, RB, C), lambda i: (0, i, 0)),
            pl.BlockSpec((NC, RB, 1), lambda i: (0, i, 0)),
            pl.BlockSpec((C, C), lambda i: (0, 0)),
            pl.BlockSpec((1, C), lambda i: (0, 0)),
        ],
        out_specs=[pl.BlockSpec((1, C), lambda i: (0, 0)),
                   pl.BlockSpec((1, C), lambda i: (0, 0))],
        out_shape=[jax.ShapeDtypeStruct((1, C), jnp.float32),
                   jax.ShapeDtypeStruct((1, C), jnp.float32)],
        scratch_shapes=[pltpu.VMEM((2, C), jnp.float32)],
    )(p, cnt, w, b)


def _decode_body(mu_ref, lv_ref, eps_ref, w_ref, b_ref, c_ref, o_ref):
    std = jnp.exp(0.5 * lv_ref[...])
    z = mu_ref[...] + eps_ref[...] * std
    r1 = jnp.tanh(jnp.dot(z, w_ref[...],
                          preferred_element_type=jnp.float32) + b_ref[...])
    r0 = jnp.tanh(b_ref[...])
    c = c_ref[0] + c_ref[1]
    o_ref[...] = jnp.where(c > 0.0, r1, r0)


def _decode(mu, lv, eps, w, b, cnt):
    return pl.pallas_call(
        _decode_body,
        grid=(NRB,),
        in_specs=[
            pl.BlockSpec((1, C), lambda i: (0, 0)),
            pl.BlockSpec((1, C), lambda i: (0, 0)),
            pl.BlockSpec((1, C), lambda i: (0, 0)),
            pl.BlockSpec((C, C), lambda i: (0, 0)),
            pl.BlockSpec((1, C), lambda i: (0, 0)),
            pl.BlockSpec((NC, RB, 1), lambda i: (0, i, 0)),
        ],
        out_specs=pl.BlockSpec((RB, C), lambda i: (i, 0)),
        out_shape=jax.ShapeDtypeStruct((N, C), jnp.float32),
    )(mu, lv, eps, w, b, cnt)


def kernel(x, edge_index, W1, b1, W2, b2, W3, b3):
    ei = edge_index.reshape(2, NW, NB, BLK).transpose(1, 2, 0, 3)
    p1, cnt_p = _make_seg_sum_sc(True)(x, ei)
    cnt = cnt_p.reshape(NC, CNT_PAD, 1)
    h1 = _combine_matmul(p1, cnt, W1, b1.reshape(1, C))
    (p2,) = _make_seg_sum_sc(False)(h1, ei)
    mu2d, lv2d = _h2_stats(p2, cnt, W2, b2.reshape(1, C))
    eps = jax.random.normal(jax.random.key(42), (1, C), dtype=jnp.float32)
    recon = _decode(mu2d, lv2d, eps, W3, b3.reshape(1, C), cnt)
    return (recon, mu2d.reshape(C), lv2d.reshape(C))


# speedup vs baseline: 1.0562x; 1.0562x over previous
import functools

import jax
import jax.numpy as jnp
from jax import lax
from jax.experimental import pallas as pl
from jax.experimental.pallas import tpu as pltpu
from jax.experimental.pallas import tpu_sc as plsc

N = 10000
E = 320000
C = 128

NC = 2
NS = 16
NW = NC * NS
EPW = E // NW
BLK = 80
NB = EPW // BLK
NPAD = 10112
RPT = NPAD // NS
ZR = 8
CNT_PAD = 10240
CSTRIPE = CNT_PAD // NS


def _seg_mean_sc_body(with_counts, feat, ei, out, cnt_out, idx_v,
                      rows_v, zrow_v, ones_v, zc_v, acc_sh, cnt_sh,
                      sem_i, sem_g, sem_s, sem_c):
    cid = lax.axis_index("c")
    sid = lax.axis_index("s")
    wid = sid * NC + cid

    def zero_zrow(k, _):
        i = k // 8
        j = k - i * 8
        zrow_v[i, pl.ds(j * 16, 16)] = jnp.zeros((16,), jnp.float32)
        return 0
    lax.fori_loop(0, ZR * 8, zero_zrow, 0)

    def zero_stripe(k, _):
        pltpu.sync_copy(zrow_v, acc_sh.at[pl.ds(sid * RPT + k * ZR, ZR), :])
        return 0
    lax.fori_loop(0, RPT // ZR, zero_stripe, 0)
    if with_counts:
        def set_ones(j, _):
            ones_v[pl.ds(j * 16, 16)] = jnp.ones((16,), jnp.float32)
            return 0
        lax.fori_loop(0, BLK // 16, set_ones, 0)

        def zero_zc(j, _):
            zc_v[pl.ds(j * 16, 16)] = jnp.zeros((16,), jnp.float32)
            return 0
        lax.fori_loop(0, CSTRIPE // 16, zero_zc, 0)
        pltpu.sync_copy(zc_v, cnt_sh.at[pl.ds(sid * CSTRIPE, CSTRIPE)])
    plsc.subcore_barrier()

    def start_idx(j, s):
        pltpu.async_copy(ei.at[wid, j], idx_v.at[s], sem_i.at[s])

    def wait_idx(j, s):
        pltpu.make_async_copy(ei.at[wid, j], idx_v.at[s],
                              sem_i.at[s]).wait()

    def start_gather(b, s):
        pltpu.async_copy(feat.at[idx_v.at[s, 0]], rows_v.at[b], sem_g.at[b])

    def wait_gather(b, s):
        pltpu.make_async_copy(feat.at[idx_v.at[s, 0]], rows_v.at[b],
                              sem_g.at[b]).wait()

    def start_scatter(b, s):
        pltpu.async_copy(rows_v.at[b], acc_sh.at[idx_v.at[s, 1]],
                         sem_s.at[b], add=True)
        if with_counts:
            pltpu.async_copy(ones_v, cnt_sh.at[idx_v.at[s, 1]],
                             sem_c.at[b], add=True)

    def wait_scatter(b, s):
        pltpu.make_async_copy(rows_v.at[b], acc_sh.at[idx_v.at[s, 1]],
                              sem_s.at[b]).wait()
        if with_counts:
            pltpu.make_async_copy(ones_v, cnt_sh.at[idx_v.at[s, 1]],
                                  sem_c.at[b]).wait()

    for t in range(7):
        start_idx(t, t)
    for t in range(4):
        wait_idx(t, t)
        start_gather(t, t)
    wait_gather(0, 0)
    start_scatter(0, 0)

    def edge_oct(k, _):
        for u in range(8):
            j = 8 * k + 1 + u
            s = (1 + u) % 8
            b = (1 + u) % 4
            wait_scatter((b - 1) % 4, (s - 1) % 8)
            pltpu.async_copy(ei.at[wid, jnp.minimum(j + 6, NB - 1)],
                             idx_v.at[(s + 6) % 8], sem_i.at[(s + 6) % 8])
            wait_idx(j + 3, (s + 3) % 8)
            start_gather((b + 3) % 4, (s + 3) % 8)
            wait_gather(b, s)
            start_scatter(b, s)
        return 0
    assert (NB - 5) % 8 == 0
    lax.fori_loop(0, (NB - 5) // 8, edge_oct, 0)
    wait_scatter(0, 0)
    wait_idx(NB - 1, 4)
    start_gather(0, 4)
    wait_gather(1, 1)
    start_scatter(1, 1)
    wait_scatter(1, 1)
    wait_gather(2, 2)
    start_scatter(2, 2)
    wait_scatter(2, 2)
    wait_gather(3, 3)
    start_scatter(3, 3)
    wait_scatter(3, 3)
    wait_gather(0, 4)
    start_scatter(0, 4)
    wait_scatter(0, 4)
    wait_idx(NB - 1, 5)
    wait_idx(NB - 1, 6)
    plsc.subcore_barrier()

    pltpu.sync_copy(acc_sh.at[pl.ds(sid * RPT, RPT), :],
                    out.at[cid, pl.ds(sid * RPT, RPT), :])
    if with_counts:
        pltpu.sync_copy(cnt_sh.at[pl.ds(sid * CSTRIPE, CSTRIPE)],
                        cnt_out.at[cid, pl.ds(sid * CSTRIPE, CSTRIPE)])


@functools.lru_cache(maxsize=None)
def _make_seg_sum_sc(with_counts):
    mesh = plsc.VectorSubcoreMesh(core_axis_name="c", subcore_axis_name="s",
                                  num_cores=NC, num_subcores=NS)
    out_type = [jax.ShapeDtypeStruct((NC, NPAD, C), jnp.float32)]
    if with_counts:
        out_type.append(jax.ShapeDtypeStruct((NC, CNT_PAD), jnp.float32))

    def body(*refs):
        if with_counts:
            feat, ei, out, cnt_out, idx_v, rows_v, zrow_v, ones_v, \
                zc_v, acc_sh, cnt_sh, sem_i, sem_g, sem_s, sem_c = refs
        else:
            feat, ei, out, idx_v, rows_v, zrow_v, acc_sh, \
                sem_i, sem_g, sem_s = refs
            cnt_out = ones_v = zc_v = cnt_sh = sem_c = None
        _seg_mean_sc_body(with_counts, feat, ei, out, cnt_out, idx_v,
                          rows_v, zrow_v, ones_v, zc_v, acc_sh, cnt_sh,
                          sem_i, sem_g, sem_s, sem_c)

    scratch = [
        pltpu.VMEM((8, 2, BLK), jnp.int32),
        pltpu.VMEM((4, BLK, C), jnp.float32),
        pltpu.VMEM((ZR, C), jnp.float32),
    ]
    if with_counts:
        scratch.append(pltpu.VMEM((BLK,), jnp.float32))
        scratch.append(pltpu.VMEM((CSTRIPE,), jnp.float32))
    scratch.append(pltpu.VMEM_SHARED((NPAD, C), jnp.float32))
    if with_counts:
        scratch.append(pltpu.VMEM_SHARED((CNT_PAD,), jnp.float32))
    scratch.append(pltpu.SemaphoreType.DMA((8,)))
    scratch.append(pltpu.SemaphoreType.DMA((4,)))
    scratch.append(pltpu.SemaphoreType.DMA((4,)))
    if with_counts:
        scratch.append(pltpu.SemaphoreType.DMA((4,)))

    return pl.kernel(body, out_type=tuple(out_type), mesh=mesh,
                     scratch_types=tuple(scratch),
                     name=f"seg_sum_sc{'_cnt' if with_counts else ''}")


RB = 1000
NRB = N // RB


def _combine_matmul_body(p_ref, c_ref, w_ref, b_ref, o_ref):
    s = p_ref[0] + p_ref[1]
    c = c_ref[0] + c_ref[1]
    agg = s / jnp.maximum(c, 1.0)
    o_ref[...] = jnp.dot(agg, w_ref[...],
                         preferred_element_type=jnp.float32) + b_ref[...]


def _combine_matmul(p, cnt, w, b):
    return pl.pallas_call(
        _combine_matmul_body,
        grid=(NRB,),
        in_specs=[
            pl.BlockSpec((NC, RB, C), lambda i: (0, i, 0)),
            pl.BlockSpec((NC, RB, 1), lambda i: (0, i, 0)),
            pl.BlockSpec((C, C), lambda i: (0, 0)),
            pl.BlockSpec((1, C), lambda i: (0, 0)),
        ],
        out_specs=pl.BlockSpec((RB, C), lambda i: (i, 0)),
        out_shape=jax.ShapeDtypeStruct((N, C), jnp.float32),
    )(p, cnt, w, b)


def _stats_body(p_ref, c_ref, w_ref, b_ref, mu_ref, var_ref, acc):
    i = pl.program_id(0)

    @pl.when(i == 0)
    def _():
        acc[...] = jnp.zeros_like(acc)

    s = p_ref[0] + p_ref[1]
    c = c_ref[0] + c_ref[1]
    agg = s / jnp.maximum(c, 1.0)
    h = jnp.dot(agg, w_ref[...], preferred_element_type=jnp.float32) \
        + b_ref[...]
    acc[0:1, :] += jnp.sum(h, axis=0, keepdims=True)
    acc[1:2, :] += jnp.sum(h * h, axis=0, keepdims=True)

    @pl.when(i == NRB - 1)
    def _():
        mu = acc[0:1, :] * (1.0 / N)
        mu_ref[...] = mu
        var_ref[...] = acc[1:2, :] * (1.0 / N) - mu * mu


def _h2_stats(p, cnt, w, b):
    return pl.pallas_call(
        _stats_body,
        grid=(NRB,),
        in_specs=[
            pl.BlockSpec((NC, RB, C), lambda i: (0, i, 0)),
            pl.BlockSpec((NC, RB, 1), lambda i: (0, i, 0)),
            pl.BlockSpec((C, C), lambda i: (0, 0)),
            pl.BlockSpec((1, C), lambda i: (0, 0)),
        ],
        out_specs=[pl.BlockSpec((1, C), lambda i: (0, 0)),
                   pl.BlockSpec((1, C), lambda i: (0, 0))],
        out_shape=[jax.ShapeDtypeStruct((1, C), jnp.float32),
                   jax.ShapeDtypeStruct((1, C), jnp.float32)],
        scratch_shapes=[pltpu.VMEM((2, C), jnp.float32)],
    )(p, cnt, w, b)


def _decode_body(mu_ref, lv_ref, eps_ref, w_ref, b_ref, c_ref, o_ref):
    std = jnp.exp(0.5 * lv_ref[...])
    z = mu_ref[...] + eps_ref[...] * std
    r1 = jnp.tanh(jnp.dot(z, w_ref[...],
                          preferred_element_type=jnp.float32) + b_ref[...])
    r0 = jnp.tanh(b_ref[...])
    c = c_ref[0] + c_ref[1]
    o_ref[...] = jnp.where(c > 0.0, r1, r0)


def _decode(mu, lv, eps, w, b, cnt):
    return pl.pallas_call(
        _decode_body,
        grid=(NRB,),
        in_specs=[
            pl.BlockSpec((1, C), lambda i: (0, 0)),
            pl.BlockSpec((1, C), lambda i: (0, 0)),
            pl.BlockSpec((1, C), lambda i: (0, 0)),
            pl.BlockSpec((C, C), lambda i: (0, 0)),
            pl.BlockSpec((1, C), lambda i: (0, 0)),
            pl.BlockSpec((NC, RB, 1), lambda i: (0, i, 0)),
        ],
        out_specs=pl.BlockSpec((RB, C), lambda i: (i, 0)),
        out_shape=jax.ShapeDtypeStruct((N, C), jnp.float32),
    )(mu, lv, eps, w, b, cnt)


def kernel(x, edge_index, W1, b1, W2, b2, W3, b3):
    ei = edge_index.reshape(2, NW, NB, BLK).transpose(1, 2, 0, 3)
    p1, cnt_p = _make_seg_sum_sc(True)(x, ei)
    cnt = cnt_p.reshape(NC, CNT_PAD, 1)
    h1 = _combine_matmul(p1, cnt, W1, b1.reshape(1, C))
    (p2,) = _make_seg_sum_sc(False)(h1, ei)
    mu2d, lv2d = _h2_stats(p2, cnt, W2, b2.reshape(1, C))
    eps = jax.random.normal(jax.random.key(42), (1, C), dtype=jnp.float32)
    recon = _decode(mu2d, lv2d, eps, W3, b3.reshape(1, C), cnt)
    return (recon, mu2d.reshape(C), lv2d.reshape(C))
